# baseline (speedup 1.0000x reference)
import jax
import jax.numpy as jnp
from jax.experimental import pallas as pl
from jax.experimental.pallas import tpu as pltpu

N = 8192
NFEAT = 128
NEMBED = 256
H1 = 256
H2 = 128
OUT = 64

BM = 512


def _leaky(v):
    return jnp.where(v >= 0.0, v, 0.01 * v)


def _body(a_ref, xf_ref, wx_ref, wn_ref,
          w1_ref, b1_ref, w2_ref, b2_ref, w3_ref, b3_ref, o_ref):
    i = pl.program_id(0)
    a = a_ref[...]
    deg = jnp.sum(a, axis=1, keepdims=True) + 1.0
    neigh = jnp.dot(a, xf_ref[...],
                    preferred_element_type=jnp.float32) / deg
    h = jnp.dot(xf_ref[pl.ds(i * BM, BM), :], wx_ref[...],
                preferred_element_type=jnp.float32)
    h += jnp.dot(neigh, wn_ref[...], preferred_element_type=jnp.float32)
    h = jnp.maximum(h, 0.0)
    h = _leaky(jnp.dot(h, w1_ref[...], preferred_element_type=jnp.float32)
               + b1_ref[...])
    h = _leaky(jnp.dot(h, w2_ref[...], preferred_element_type=jnp.float32)
               + b2_ref[...])
    h = _leaky(jnp.dot(h, w3_ref[...], preferred_element_type=jnp.float32)
               + b3_ref[...])
    m = jnp.max(h, axis=1, keepdims=True)
    e = jnp.exp(h - m)
    o_ref[pl.ds(i * BM, BM), :] = e / jnp.sum(e, axis=1, keepdims=True)


@jax.jit
def kernel(x, adj, W_sage, W1, b1, W2, b2, W3, b3):
    wx = W_sage[:, :NFEAT].T
    wn = W_sage[:, NFEAT:].T
    w1t, w2t, w3t = W1.T, W2.T, W3.T
    b1r = b1.reshape(1, H1)
    b2r = b2.reshape(1, H2)
    b3r = b3.reshape(1, OUT)

    grid = (N // BM,)
    whole = lambda r, c: pl.BlockSpec((r, c), lambda i: (0, 0))
    out = pl.pallas_call(
        _body,
        grid=grid,
        in_specs=[
            pl.BlockSpec((BM, N), lambda i: (i, 0)),
            whole(N, NFEAT),
            whole(NFEAT, NEMBED),
            whole(NFEAT, NEMBED),
            whole(NEMBED, H1),
            whole(1, H1),
            whole(H1, H2),
            whole(1, H2),
            whole(H2, OUT),
            whole(1, OUT),
        ],
        out_specs=pl.BlockSpec((N, OUT), lambda i: (0, 0)),
        out_shape=jax.ShapeDtypeStruct((N, OUT), jnp.float32),
        compiler_params=pltpu.CompilerParams(
            dimension_semantics=("parallel",),
        ),
    )(adj, x, wx, wn, w1t, b1r, w2t, b2r, w3t, b3r)
    return out


# speedup vs baseline: 1.0025x; 1.0025x over previous
import jax
import jax.numpy as jnp
from jax.experimental import pallas as pl
from jax.experimental.pallas import tpu as pltpu

N = 8192
NFEAT = 128
NEMBED = 256
H1 = 256
H2 = 128
OUT = 64

BM = 512


def _leaky(v):
    return jnp.where(v >= 0.0, v, 0.01 * v)


def _body(a_ref, xf_ref, wx_ref, wn_ref,
          w1_ref, b1_ref, w2_ref, b2_ref, w3_ref, b3_ref, o_ref):
    i = pl.program_id(0)
    a = a_ref[...]
    deg = jnp.sum(a, axis=1, keepdims=True) + 1.0
    neigh = jnp.dot(a, xf_ref[...],
                    preferred_element_type=jnp.float32) / deg
    h = jnp.dot(xf_ref[pl.ds(i * BM, BM), :], wx_ref[...],
                preferred_element_type=jnp.float32)
    h += jnp.dot(neigh, wn_ref[...], preferred_element_type=jnp.float32)
    h = jnp.maximum(h, 0.0)
    h = _leaky(jnp.dot(h, w1_ref[...], preferred_element_type=jnp.float32)
               + b1_ref[...])
    h = _leaky(jnp.dot(h, w2_ref[...], preferred_element_type=jnp.float32)
               + b2_ref[...])
    h = _leaky(jnp.dot(h, w3_ref[...], preferred_element_type=jnp.float32)
               + b3_ref[...])
    m = jnp.max(h, axis=1, keepdims=True)
    e = jnp.exp(h - m)
    o_ref[...] = e / jnp.sum(e, axis=1, keepdims=True)


@jax.jit
def kernel(x, adj, W_sage, W1, b1, W2, b2, W3, b3):
    wx = W_sage[:, :NFEAT].T
    wn = W_sage[:, NFEAT:].T
    w1t, w2t, w3t = W1.T, W2.T, W3.T
    b1r = b1.reshape(1, H1)
    b2r = b2.reshape(1, H2)
    b3r = b3.reshape(1, OUT)

    grid = (N // BM,)
    whole = lambda r, c: pl.BlockSpec((r, c), lambda i: (0, 0))
    out = pl.pallas_call(
        _body,
        grid=grid,
        in_specs=[
            pl.BlockSpec((BM, N), lambda i: (i, 0)),
            whole(N, NFEAT),
            whole(NFEAT, NEMBED),
            whole(NFEAT, NEMBED),
            whole(NEMBED, H1),
            whole(1, H1),
            whole(H1, H2),
            whole(1, H2),
            whole(H2, OUT),
            whole(1, OUT),
        ],
        out_specs=pl.BlockSpec((BM, OUT), lambda i: (i, 0)),
        out_shape=jax.ShapeDtypeStruct((N, OUT), jnp.float32),
        compiler_params=pltpu.CompilerParams(
            dimension_semantics=("parallel",),
        ),
    )(adj, x, wx, wn, w1t, b1r, w2t, b2r, w3t, b3r)
    return out
